# Initial kernel scaffold; baseline (speedup 1.0000x reference)
import functools

import jax
import jax.numpy as jnp
from jax import lax
from jax.experimental import pallas as pl
from jax.experimental.pallas import tpu as pltpu
from jax.experimental.pallas import tpu_sc as plsc

F32 = jnp.float32

NODE_BLK = 2000
EDGE_BLK = 512
CH = 128


def _node_body(xj_ref, xi_ref, wj_ref, wi_ref, oj_ref, oi_ref):
    oj_ref[...] = jnp.dot(xj_ref[...], wj_ref[...], preferred_element_type=F32)
    oi_ref[...] = jnp.dot(xi_ref[...], wi_ref[...], preferred_element_type=F32)


def _node_transform(x_j, x_i, w_j, w_i):
    n, d = x_j.shape
    grid = (n + NODE_BLK - 1) // NODE_BLK
    return pl.pallas_call(
        _node_body,
        grid=(grid,),
        in_specs=[
            pl.BlockSpec((NODE_BLK, d), lambda i: (i, 0)),
            pl.BlockSpec((NODE_BLK, d), lambda i: (i, 0)),
            pl.BlockSpec((d, d), lambda i: (0, 0)),
            pl.BlockSpec((d, d), lambda i: (0, 0)),
        ],
        out_specs=[
            pl.BlockSpec((NODE_BLK, d), lambda i: (i, 0)),
            pl.BlockSpec((NODE_BLK, d), lambda i: (i, 0)),
        ],
        out_shape=[
            jax.ShapeDtypeStruct((n, d), F32),
            jax.ShapeDtypeStruct((n, d), F32),
        ],
    )(x_j, x_i, w_j, w_i)


def _rels_body(rels_ref, pa1_ref, pw1_ref, pb1_ref, pa2_ref, pw2_ref, pb2_ref,
               av_ref, mw2_ref, mb2_ref, r_ref, v_ref, c_ref):
    pa1 = pa1_ref[0, 0]
    pa2 = pa2_ref[0, 0]
    r = rels_ref[...]
    r = jnp.where(r >= 0, r, pa1 * r)
    r = lax.dot_general(r, pw1_ref[...], (((1,), (1,)), ((), ())),
                        preferred_element_type=F32) + pb1_ref[...]
    r = jnp.where(r >= 0, r, pa2 * r)
    r = lax.dot_general(r, pw2_ref[...], (((1,), (1,)), ((), ())),
                        preferred_element_type=F32) + pb2_ref[...]
    r_ref[...] = r
    v_ref[...] = lax.dot_general(av_ref[...], mw2_ref[...],
                                 (((1,), (0,)), ((), ())),
                                 preferred_element_type=F32)
    c_ref[...] = lax.dot_general(av_ref[...], mb2_ref[...],
                                 (((1,), (1,)), ((), ())),
                                 preferred_element_type=F32)


def _rels_transform(rels, pa1, pw1, pb1, pa2, pw2, pb2, a, mw2, mb2):
    b, d = rels.shape
    return pl.pallas_call(
        _rels_body,
        out_shape=[
            jax.ShapeDtypeStruct((b, d), F32),
            jax.ShapeDtypeStruct((1, d), F32),
            jax.ShapeDtypeStruct((1, 1), F32),
        ],
    )(rels, pa1.reshape(1, 1), pw1, pb1.reshape(1, d), pa2.reshape(1, 1),
      pw2, pb2.reshape(1, d), a.reshape(1, d), mw2, mb2.reshape(1, d))


def _sc_gather_sum(xjt, xit, r, src, dst, bat, a1vec, e_pad, d):
    info = plsc.get_sparse_core_info()
    nw = info.num_cores * info.num_subcores
    epw = e_pad // nw
    nch = epw // CH
    nvec = CH * d // 16
    mesh = plsc.VectorSubcoreMesh(core_axis_name="c", subcore_axis_name="s")

    @functools.partial(
        pl.kernel,
        out_type=jax.ShapeDtypeStruct((e_pad, d), F32),
        mesh=mesh,
        scratch_types=[
            pltpu.VMEM((CH,), jnp.int32),
            pltpu.VMEM((CH,), jnp.int32),
            pltpu.VMEM((CH,), jnp.int32),
            pltpu.VMEM((CH, d), F32),
            pltpu.VMEM((CH, d), F32),
            pltpu.VMEM((CH, d), F32),
            pltpu.VMEM((16,), F32),
            pltpu.SemaphoreType.DMA,
        ],
    )
    def k2(xjt_hbm, xit_hbm, r_hbm, src_hbm, dst_hbm, bat_hbm, a1_hbm,
           out_hbm, isv, idv, ibv, bufa, bufb, bufc, a1v, sem):
        wid = lax.axis_index("s") * info.num_cores + lax.axis_index("c")
        pltpu.sync_copy(a1_hbm, a1v)
        a1 = a1v[...]

        def chunk(j, _):
            base = wid * epw + j * CH
            pltpu.sync_copy(src_hbm.at[pl.ds(base, CH)], isv)
            pltpu.sync_copy(dst_hbm.at[pl.ds(base, CH)], idv)
            pltpu.sync_copy(bat_hbm.at[pl.ds(base, CH)], ibv)
            cpa = pltpu.async_copy(xjt_hbm.at[isv], bufa, sem)
            cpb = pltpu.async_copy(xit_hbm.at[idv], bufb, sem)
            cpc = pltpu.async_copy(r_hbm.at[ibv], bufc, sem)
            cpa.wait()
            cpb.wait()
            cpc.wait()

            def vec(i, _):
                row = i // (d // 16)
                col = (i % (d // 16)) * 16
                x = (bufa[row, pl.ds(col, 16)] + bufb[row, pl.ds(col, 16)]
                     + bufc[row, pl.ds(col, 16)])
                bufa[row, pl.ds(col, 16)] = jnp.where(x >= 0, x, a1 * x)
                return 0

            lax.fori_loop(0, nvec, vec, 0, unroll=8)
            pltpu.sync_copy(bufa, out_hbm.at[pl.ds(base, CH)])
            return 0

        lax.fori_loop(0, nch, chunk, 0)

    return k2(xjt, xit, r, src, dst, bat, a1vec)


def _edge_body(e_total, b, h1_ref, w1_ref, b1_ref, a2_ref, v_ref, c_ref,
               bid_ref, e_ref, s_ref):
    pid = pl.program_id(0)
    a2 = a2_ref[0, 0]
    z = lax.dot_general(h1_ref[...], w1_ref[...], (((1,), (1,)), ((), ())),
                        preferred_element_type=F32) + b1_ref[...]
    z = jnp.where(z >= 0, z, a2 * z)
    alpha = lax.dot_general(z, v_ref[...], (((1,), (1,)), ((), ())),
                            preferred_element_type=F32) + c_ref[...]
    gidx = pid * EDGE_BLK + lax.broadcasted_iota(jnp.int32, (EDGE_BLK, 1), 0)
    e = jnp.where(gidx < e_total, jnp.exp(alpha), 0.0)
    e_ref[...] = e
    bid = bid_ref[0, 0, :].reshape(EDGE_BLK, 1)
    onehot = (bid == lax.broadcasted_iota(jnp.int32, (EDGE_BLK, b), 1)
              ).astype(F32)
    spart = lax.dot_general(e, onehot, (((0,), (0,)), ((), ())),
                            preferred_element_type=F32)

    @pl.when(pid == 0)
    def _():
        s_ref[...] = jnp.zeros_like(s_ref)

    s_ref[...] += spart


def _edge_mlp(h1, w1, b1, a2, v, c, bid3, e_total, b):
    e_pad, d = h1.shape
    grid = e_pad // EDGE_BLK
    return pl.pallas_call(
        functools.partial(_edge_body, e_total, b),
        grid=(grid,),
        in_specs=[
            pl.BlockSpec((EDGE_BLK, d), lambda i: (i, 0)),
            pl.BlockSpec((d, d), lambda i: (0, 0)),
            pl.BlockSpec((1, d), lambda i: (0, 0)),
            pl.BlockSpec((1, 1), lambda i: (0, 0)),
            pl.BlockSpec((1, d), lambda i: (0, 0)),
            pl.BlockSpec((1, 1), lambda i: (0, 0)),
            pl.BlockSpec((1, 1, EDGE_BLK), lambda i: (i, 0, 0)),
        ],
        out_specs=[
            pl.BlockSpec((EDGE_BLK, 1), lambda i: (i, 0)),
            pl.BlockSpec((1, b), lambda i: (0, 0)),
        ],
        out_shape=[
            jax.ShapeDtypeStruct((e_pad, 1), F32),
            jax.ShapeDtypeStruct((1, b), F32),
        ],
    )(h1, w1, b1, a2, v, c, bid3)


def _sc_divide(e_flat, s_flat, bat, e_pad, b):
    info = plsc.get_sparse_core_info()
    nw = info.num_cores * info.num_subcores
    epw = e_pad // nw
    nch = epw // CH
    mesh = plsc.VectorSubcoreMesh(core_axis_name="c", subcore_axis_name="s")

    @functools.partial(
        pl.kernel,
        out_type=jax.ShapeDtypeStruct((e_pad,), F32),
        mesh=mesh,
        scratch_types=[
            pltpu.VMEM((b,), F32),
            pltpu.VMEM((CH,), F32),
            pltpu.VMEM((CH,), jnp.int32),
            pltpu.VMEM((CH,), F32),
        ],
    )
    def k4(e_hbm, s_hbm, bat_hbm, out_hbm, sv, ev, bv, ov):
        wid = lax.axis_index("s") * info.num_cores + lax.axis_index("c")
        pltpu.sync_copy(s_hbm, sv)

        def chunk(j, _):
            base = wid * epw + j * CH
            pltpu.sync_copy(e_hbm.at[pl.ds(base, CH)], ev)
            pltpu.sync_copy(bat_hbm.at[pl.ds(base, CH)], bv)

            def vec(i, _):
                off = i * 16
                idx = bv[pl.ds(off, 16)]
                denom = plsc.load_gather(sv, [idx]) + 1e-16
                ov[pl.ds(off, 16)] = ev[pl.ds(off, 16)] / denom
                return 0

            lax.fori_loop(0, CH // 16, vec, 0, unroll=8)
            pltpu.sync_copy(ov, out_hbm.at[pl.ds(base, CH)])
            return 0

        lax.fori_loop(0, nch, chunk, 0)

    return k4(e_flat, s_flat, bat)


def kernel(x_j, x_i, rels, w_i, w_j, a, proj_a1, proj_W1, proj_b1, proj_a2,
           proj_W2, proj_b2, mlp_a1, mlp_W1, mlp_b1, mlp_a2, mlp_W2, mlp_b2,
           edge_index, edge_index_batch):
    n, d = x_j.shape
    b = rels.shape[0]
    e_total = edge_index.shape[1]
    e_pad = ((e_total + 32 * CH - 1) // (32 * CH)) * (32 * CH)
    pad = e_pad - e_total

    src = jnp.pad(edge_index[0], (0, pad))
    dst = jnp.pad(edge_index[1], (0, pad))
    bat = jnp.pad(edge_index_batch, (0, pad))

    xjt, xit = _node_transform(x_j, x_i, w_j, w_i)
    r, v, c = _rels_transform(rels, proj_a1, proj_W1, proj_b1, proj_a2,
                              proj_W2, proj_b2, a, mlp_W2, mlp_b2)
    a1vec = jnp.full((16,), mlp_a1, dtype=F32)
    h1 = _sc_gather_sum(xjt, xit, r, src, dst, bat, a1vec, e_pad, d)
    bid3 = bat.reshape(e_pad // EDGE_BLK, 1, EDGE_BLK)
    e_col, s_row = _edge_mlp(h1, mlp_W1, mlp_b1, mlp_a2.reshape(1, 1), v, c,
                             bid3, e_total, b)
    out_pad = _sc_divide(e_col.reshape(e_pad), s_row.reshape(b), bat,
                         e_pad, b)
    return out_pad[:e_total]


# speedup vs baseline: 1.7023x; 1.7023x over previous
import functools

import jax
import jax.numpy as jnp
from jax import lax
from jax.experimental import pallas as pl
from jax.experimental.pallas import tpu as pltpu
from jax.experimental.pallas import tpu_sc as plsc

F32 = jnp.float32

NODE_BLK = 2000
EDGE_BLK = 512
CH = 128


def _node_body(xj_ref, xi_ref, wj_ref, wi_ref, oj_ref, oi_ref):
    oj_ref[...] = jnp.dot(xj_ref[...], wj_ref[...], preferred_element_type=F32)
    oi_ref[...] = jnp.dot(xi_ref[...], wi_ref[...], preferred_element_type=F32)


def _node_transform(x_j, x_i, w_j, w_i):
    n, d = x_j.shape
    grid = (n + NODE_BLK - 1) // NODE_BLK
    return pl.pallas_call(
        _node_body,
        grid=(grid,),
        in_specs=[
            pl.BlockSpec((NODE_BLK, d), lambda i: (i, 0)),
            pl.BlockSpec((NODE_BLK, d), lambda i: (i, 0)),
            pl.BlockSpec((d, d), lambda i: (0, 0)),
            pl.BlockSpec((d, d), lambda i: (0, 0)),
        ],
        out_specs=[
            pl.BlockSpec((NODE_BLK, d), lambda i: (i, 0)),
            pl.BlockSpec((NODE_BLK, d), lambda i: (i, 0)),
        ],
        out_shape=[
            jax.ShapeDtypeStruct((n, d), F32),
            jax.ShapeDtypeStruct((n, d), F32),
        ],
    )(x_j, x_i, w_j, w_i)


def _rels_body(rels_ref, pa1_ref, pw1_ref, pb1_ref, pa2_ref, pw2_ref, pb2_ref,
               av_ref, mw2_ref, mb2_ref, r_ref, v_ref, c_ref):
    pa1 = pa1_ref[0, 0]
    pa2 = pa2_ref[0, 0]
    r = rels_ref[...]
    r = jnp.where(r >= 0, r, pa1 * r)
    r = lax.dot_general(r, pw1_ref[...], (((1,), (1,)), ((), ())),
                        preferred_element_type=F32) + pb1_ref[...]
    r = jnp.where(r >= 0, r, pa2 * r)
    r = lax.dot_general(r, pw2_ref[...], (((1,), (1,)), ((), ())),
                        preferred_element_type=F32) + pb2_ref[...]
    r_ref[...] = r
    v_ref[...] = lax.dot_general(av_ref[...], mw2_ref[...],
                                 (((1,), (0,)), ((), ())),
                                 preferred_element_type=F32)
    c_ref[...] = lax.dot_general(av_ref[...], mb2_ref[...],
                                 (((1,), (1,)), ((), ())),
                                 preferred_element_type=F32)


def _rels_transform(rels, pa1, pw1, pb1, pa2, pw2, pb2, a, mw2, mb2):
    b, d = rels.shape
    return pl.pallas_call(
        _rels_body,
        out_shape=[
            jax.ShapeDtypeStruct((b, d), F32),
            jax.ShapeDtypeStruct((1, d), F32),
            jax.ShapeDtypeStruct((1, 1), F32),
        ],
    )(rels, pa1.reshape(1, 1), pw1, pb1.reshape(1, d), pa2.reshape(1, 1),
      pw2, pb2.reshape(1, d), a.reshape(1, d), mw2, mb2.reshape(1, d))


def _sc_gather_sum(xjt, xit, r, src, dst, bat, a1vec, e_pad, d):
    info = plsc.get_sparse_core_info()
    nw = info.num_cores * info.num_subcores
    epw = e_pad // nw
    nch = epw // CH
    nvec = CH * d // 16
    mesh = plsc.VectorSubcoreMesh(core_axis_name="c", subcore_axis_name="s")

    @functools.partial(
        pl.kernel,
        out_type=jax.ShapeDtypeStruct((e_pad, d), F32),
        mesh=mesh,
        scratch_types=[
            pltpu.VMEM((CH,), jnp.int32),
            pltpu.VMEM((CH,), jnp.int32),
            pltpu.VMEM((CH,), jnp.int32),
            pltpu.VMEM((CH, d), F32),
            pltpu.VMEM((CH, d), F32),
            pltpu.VMEM((CH, d), F32),
            pltpu.VMEM((16,), F32),
            pltpu.SemaphoreType.DMA,
        ],
    )
    def k2(xjt_hbm, xit_hbm, r_hbm, src_hbm, dst_hbm, bat_hbm, a1_hbm,
           out_hbm, isv, idv, ibv, bufa, bufb, bufc, a1v, sem):
        wid = lax.axis_index("s") * info.num_cores + lax.axis_index("c")
        pltpu.sync_copy(a1_hbm, a1v)
        a1 = a1v[...]

        def chunk(j, _):
            base = wid * epw + j * CH
            pltpu.sync_copy(src_hbm.at[pl.ds(base, CH)], isv)
            pltpu.sync_copy(dst_hbm.at[pl.ds(base, CH)], idv)
            pltpu.sync_copy(bat_hbm.at[pl.ds(base, CH)], ibv)
            cpa = pltpu.async_copy(xjt_hbm.at[isv], bufa, sem)
            cpb = pltpu.async_copy(xit_hbm.at[idv], bufb, sem)
            cpc = pltpu.async_copy(r_hbm.at[ibv], bufc, sem)
            cpa.wait()
            cpb.wait()
            cpc.wait()

            def vec(i, _):
                row = i // (d // 16)
                col = (i % (d // 16)) * 16
                x = (bufa[row, pl.ds(col, 16)] + bufb[row, pl.ds(col, 16)]
                     + bufc[row, pl.ds(col, 16)])
                bufa[row, pl.ds(col, 16)] = jnp.where(x >= 0, x, a1 * x)
                return 0

            lax.fori_loop(0, nvec, vec, 0, unroll=8)
            pltpu.sync_copy(bufa, out_hbm.at[pl.ds(base, CH)])
            return 0

        lax.fori_loop(0, nch, chunk, 0)

    return k2(xjt, xit, r, src, dst, bat, a1vec)


def _edge_body(e_total, b, h1_ref, w1_ref, b1_ref, a2_ref, v_ref, c_ref,
               bid_ref, e_ref, s_ref):
    pid = pl.program_id(0)
    a2 = a2_ref[0, 0]
    z = lax.dot_general(h1_ref[...], w1_ref[...], (((1,), (1,)), ((), ())),
                        preferred_element_type=F32) + b1_ref[...]
    z = jnp.where(z >= 0, z, a2 * z)
    alpha = lax.dot_general(z, v_ref[...], (((1,), (0,)), ((), ())),
                            preferred_element_type=F32) + c_ref[0, 0]
    gidx = pid * EDGE_BLK + lax.broadcasted_iota(jnp.int32, (EDGE_BLK, 1), 0)
    e = jnp.where(gidx < e_total, jnp.exp(alpha), 0.0)
    e_ref[...] = e
    bid = bid_ref[...]
    onehot = (bid == lax.broadcasted_iota(jnp.int32, (EDGE_BLK, b), 1)
              ).astype(F32)
    spart = lax.dot_general(e, onehot, (((0,), (0,)), ((), ())),
                            preferred_element_type=F32)

    @pl.when(pid == 0)
    def _():
        s_ref[...] = jnp.zeros_like(s_ref)

    s_ref[...] += spart


def _edge_mlp(h1, w1, b1, a2, v, c, bid2, e_total, b):
    e_pad, d = h1.shape
    grid = e_pad // EDGE_BLK
    return pl.pallas_call(
        functools.partial(_edge_body, e_total, b),
        grid=(grid,),
        in_specs=[
            pl.BlockSpec((EDGE_BLK, d), lambda i: (i, 0)),
            pl.BlockSpec((d, d), lambda i: (0, 0)),
            pl.BlockSpec((1, d), lambda i: (0, 0)),
            pl.BlockSpec((1, 1), lambda i: (0, 0)),
            pl.BlockSpec((d, 1), lambda i: (0, 0)),
            pl.BlockSpec((1, 1), lambda i: (0, 0)),
            pl.BlockSpec((EDGE_BLK, 1), lambda i: (i, 0)),
        ],
        out_specs=[
            pl.BlockSpec((EDGE_BLK, 1), lambda i: (i, 0)),
            pl.BlockSpec((1, b), lambda i: (0, 0)),
        ],
        out_shape=[
            jax.ShapeDtypeStruct((e_pad, 1), F32),
            jax.ShapeDtypeStruct((1, b), F32),
        ],
    )(h1, w1, b1, a2, v, c, bid2)


def _norm_body(b, e_ref, s_ref, bid_ref, o_ref):
    sinv = 1.0 / (s_ref[...] + 1e-16)
    bid = bid_ref[...]
    onehot = (bid == lax.broadcasted_iota(jnp.int32, (EDGE_BLK, b), 1)
              ).astype(F32)
    se = lax.dot_general(onehot, sinv, (((1,), (1,)), ((), ())),
                         preferred_element_type=F32)
    o_ref[...] = e_ref[...] * se


def _normalize(e_col, s_row, bid2, b):
    e_pad = e_col.shape[0]
    grid = e_pad // EDGE_BLK
    return pl.pallas_call(
        functools.partial(_norm_body, b),
        grid=(grid,),
        in_specs=[
            pl.BlockSpec((EDGE_BLK, 1), lambda i: (i, 0)),
            pl.BlockSpec((1, b), lambda i: (0, 0)),
            pl.BlockSpec((EDGE_BLK, 1), lambda i: (i, 0)),
        ],
        out_specs=pl.BlockSpec((EDGE_BLK, 1), lambda i: (i, 0)),
        out_shape=jax.ShapeDtypeStruct((e_pad, 1), F32),
    )(e_col, s_row, bid2)


def kernel(x_j, x_i, rels, w_i, w_j, a, proj_a1, proj_W1, proj_b1, proj_a2,
           proj_W2, proj_b2, mlp_a1, mlp_W1, mlp_b1, mlp_a2, mlp_W2, mlp_b2,
           edge_index, edge_index_batch):
    n, d = x_j.shape
    b = rels.shape[0]
    e_total = edge_index.shape[1]
    e_pad = ((e_total + 32 * CH - 1) // (32 * CH)) * (32 * CH)
    pad = e_pad - e_total

    src = jnp.pad(edge_index[0], (0, pad))
    dst = jnp.pad(edge_index[1], (0, pad))
    bat = jnp.pad(edge_index_batch, (0, pad))

    xjt, xit = _node_transform(x_j, x_i, w_j, w_i)
    r, v, c = _rels_transform(rels, proj_a1, proj_W1, proj_b1, proj_a2,
                              proj_W2, proj_b2, a, mlp_W2, mlp_b2)
    a1vec = jnp.full((16,), mlp_a1, dtype=F32)
    h1 = _sc_gather_sum(xjt, xit, r, src, dst, bat, a1vec, e_pad, d)
    bid2 = bat.reshape(e_pad, 1)
    e_col, s_row = _edge_mlp(h1, mlp_W1, mlp_b1.reshape(1, d),
                             mlp_a2.reshape(1, 1), v.reshape(d, 1), c, bid2,
                             e_total, b)
    out_col = _normalize(e_col, s_row, bid2, b)
    return out_col.reshape(e_pad)[:e_total]
